# baseline (speedup 1.0000x reference)
import functools

import jax
import jax.numpy as jnp
from jax import lax
from jax.experimental import pallas as pl
from jax.experimental.pallas import tpu as pltpu
from jax.experimental.pallas import tpu_sc as plsc

B = 16384
F = 26
D = 32
V = 100000
N = B * F

NW = 32
RPW = N // NW
G = 128
NG = RPW // G

_mesh = plsc.VectorSubcoreMesh(core_axis_name="c", subcore_axis_name="s")


@functools.partial(
    pl.kernel,
    out_type=jax.ShapeDtypeStruct((N, D), jnp.float32),
    mesh=_mesh,
    scratch_types=[
        pltpu.VMEM((RPW,), jnp.int32),
        pltpu.VMEM((NG, G), jnp.int32),
        pltpu.VMEM((G, D), jnp.float32),
        pltpu.SemaphoreType.DMA,
    ],
    compiler_params=pltpu.CompilerParams(use_tc_tiling_on_sc=False),
)
def _sc_gather(x_hbm, tbl_hbm, out_hbm, xv, idxv, rowbuf, sem):
    wid = lax.axis_index("s") * 2 + lax.axis_index("c")
    row0 = wid * RPW

    pltpu.sync_copy(x_hbm.at[pl.ds(row0, RPW)], xv)

    iota = lax.iota(jnp.int32, 16)

    def idx_body(i, _):
        pos = row0 + i * 16 + iota
        f = lax.rem(pos, F)
        idxv[i // 8, pl.ds((i % 8) * 16, 16)] = xv[pl.ds(i * 16, 16)] * F + f
        return _

    lax.fori_loop(0, RPW // 16, idx_body, None)

    def gather_body(g, _):
        pltpu.async_copy(tbl_hbm.at[idxv.at[g]], rowbuf, sem).wait()
        pltpu.sync_copy(rowbuf, out_hbm.at[pl.ds(row0 + g * G, G)])
        return _

    lax.fori_loop(0, NG, gather_body, None)


def kernel(x, weight):
    x32 = x.astype(jnp.int32).reshape(N)
    table = lax.optimization_barrier(weight.reshape(V * F, D))
    out = _sc_gather(x32, table)
    return out.reshape(B, F * D)


# speedup vs baseline: 5.4327x; 5.4327x over previous
import functools

import jax
import jax.numpy as jnp
from jax import lax
from jax.experimental import pallas as pl
from jax.experimental.pallas import tpu as pltpu
from jax.experimental.pallas import tpu_sc as plsc

B = 16384
F = 26
D = 32
V = 100000

NW = 32
BPW = B // NW
CB = 8
NCHUNK = BPW // CB
ROWS = CB * F
XPW = BPW * F

_mesh = plsc.VectorSubcoreMesh(core_axis_name="c", subcore_axis_name="s")


@functools.partial(
    pl.kernel,
    out_type=jax.ShapeDtypeStruct((B, F * D), jnp.float32),
    mesh=_mesh,
    scratch_types=[
        pltpu.VMEM((XPW,), jnp.int32),
        pltpu.VMEM((2, 8, 32), jnp.int32),
        pltpu.VMEM((2, ROWS, 128), jnp.float32),
        pltpu.VMEM((CB, F * D), jnp.float32),
        pltpu.SemaphoreType.DMA,
    ],
    compiler_params=pltpu.CompilerParams(
        use_tc_tiling_on_sc=True, needs_layout_passes=False),
)
def _sc_gather(x_hbm, w_hbm, w2_hbm, out_hbm, xv, widx, gbuf, outbuf, sem):
    wid = lax.axis_index("s") * 2 + lax.axis_index("c")
    pltpu.sync_copy(x_hbm.at[pl.ds(wid * XPW, XPW)], xv)

    iota = lax.iota(jnp.int32, 16)
    pat4 = (iota // 4) * F + (iota % 4)
    pat2 = (iota // 2) * F + (iota % 2)

    def build_and_fire(c, p):
        p0 = c * ROWS
        for w in range(6):
            for t in range(2):
                src = pat4 + (p0 + 104 * t + 4 * w)
                widx[p, w, pl.ds(16 * t, 16)] = plsc.load_gather(xv, [src])
        widx[p, 6, pl.ds(0, 16)] = plsc.load_gather(xv, [pat2 + (p0 + 24)])
        for w in range(6):
            pltpu.async_copy(
                w_hbm.at[widx.at[p, w], pl.ds(128 * w, 128)],
                gbuf.at[p, pl.ds(32 * w, 32)], sem)
        pltpu.async_copy(
            w2_hbm.at[widx.at[p, 6, pl.ds(0, 16)]],
            gbuf.at[p, pl.ds(192, 16)], sem)

    build_and_fire(0, 0)

    def chunk_body(c, carry):
        p = lax.rem(c, 2)

        @pl.when(c + 1 < NCHUNK)
        def _():
            build_and_fire(c + 1, lax.rem(c + 1, 2))

        pltpu.make_async_copy(
            w_hbm.at[pl.ds(0, ROWS), pl.ds(0, 128)], gbuf.at[p], sem).wait()

        def ext_body(b, _):
            for f in range(24):
                src = 32 * (f // 4) + b * 4 + (f % 4)
                off = 32 * (f % 4)
                outbuf[b, pl.ds(32 * f, 16)] = gbuf[p, src, pl.ds(off, 16)]
                outbuf[b, pl.ds(32 * f + 16, 16)] = gbuf[p, src, pl.ds(off + 16, 16)]
            for f in range(24, F):
                src = 192 + b * 2 + (f - 24)
                off = 64 + 32 * (f - 24)
                outbuf[b, pl.ds(32 * f, 16)] = gbuf[p, src, pl.ds(off, 16)]
                outbuf[b, pl.ds(32 * f + 16, 16)] = gbuf[p, src, pl.ds(off + 16, 16)]
            return _

        lax.fori_loop(0, CB, ext_body, None)
        pltpu.sync_copy(outbuf, out_hbm.at[pl.ds(wid * BPW + c * CB, CB)])
        return carry

    lax.fori_loop(0, NCHUNK, chunk_body, None)


def kernel(x, weight):
    x32 = x.astype(jnp.int32).reshape(B * F)
    w2 = lax.slice(weight, (0, 704), (V, 832))
    return _sc_gather(x32, weight, w2)
